# baseline (speedup 1.0000x reference)
import jax
import jax.numpy as jnp
from jax import lax
from jax.experimental import pallas as pl
from jax.experimental.pallas import tpu as pltpu
from jax.experimental.pallas import tpu_sc as plsc

NU = 50000
H = 32
E = 800000
NC, NS, LN = 2, 16, 16
CL = 128
EP = 802816
G = EP // NS // CL
J = 8
NG = G // J
R = 51200
RT = R // NS
TRASH = R - 1
NBUF = 4

_mesh = plsc.VectorSubcoreMesh(
    core_axis_name="c", subcore_axis_name="s", num_cores=NC, num_subcores=NS
)
_params = pltpu.CompilerParams(use_tc_tiling_on_sc=False)


def _zero_f32(ref, nrows, ncols):
    z = jnp.zeros((LN,), jnp.float32)

    def body(r, _):
        for c0 in range(0, ncols, LN):
            ref[r, pl.ds(c0, LN)] = z
        return 0

    lax.fori_loop(0, nrows, body, 0)


def _zero_f32_1d(ref, n):
    z = jnp.zeros((LN,), jnp.float32)

    def body(r, _):
        ref[pl.ds(r * LN, LN)] = z
        return 0

    lax.fori_loop(0, n // LN, body, 0)


def _hist_body(eidx, cnt, didx, ones, zbuf, acc):
    c = lax.axis_index("c")
    s = lax.axis_index("s")
    one = jnp.ones((LN,), jnp.float32)
    for c0 in range(0, CL, LN):
        ones[pl.ds(c0, LN)] = one
    _zero_f32_1d(zbuf, RT)
    pltpu.sync_copy(zbuf, acc.at[pl.ds(s * RT, RT)])
    plsc.subcore_barrier()

    def group(g, _):
        base = s * G + g * J
        pltpu.sync_copy(eidx.at[c, pl.ds(base, J)], didx)
        for j in range(J):
            pltpu.sync_copy(ones, acc.at[didx.at[j]], add=True)
        return 0

    lax.fori_loop(0, NG, group, 0)
    plsc.subcore_barrier()
    pltpu.sync_copy(acc.at[pl.ds(s * RT, RT)], cnt.at[c, pl.ds(s * RT, RT)])


_hist = pl.kernel(
    _hist_body,
    out_type=jax.ShapeDtypeStruct((NC, R), jnp.float32),
    mesh=_mesh,
    scratch_types=[
        pltpu.VMEM((J, CL), jnp.int32),
        pltpu.VMEM((CL,), jnp.float32),
        pltpu.VMEM((RT,), jnp.float32),
        pltpu.VMEM_SHARED((R,), jnp.float32),
    ],
    compiler_params=_params,
)


def _layer_body(eidx, w0, w1, o0, o1, sidx, didx, zbuf, acc, *rows_sems):
    rows = rows_sems[:NBUF]
    gsem = rows_sems[NBUF:]
    c = lax.axis_index("c")
    s = lax.axis_index("s")
    notc = 1 - c
    _zero_f32(zbuf, CL, H)
    for h, (w, o) in enumerate(((w0, o0), (w1, o1))):
        ytab = w.at[notc]
        for k in range(RT // CL):
            pltpu.sync_copy(zbuf, acc.at[pl.ds(s * RT + k * CL, CL)])
        plsc.subcore_barrier()

        def group(g, _):
            base = s * G + g * J
            pltpu.sync_copy(eidx.at[notc, pl.ds(base, J)], sidx)
            pltpu.sync_copy(eidx.at[c, pl.ds(base, J)], didx)
            for b in range(NBUF):
                pltpu.async_copy(ytab.at[sidx.at[b]], rows[b], gsem[b])
            for j in range(J):
                b = j % NBUF
                pltpu.make_async_copy(ytab.at[sidx.at[j]], rows[b],
                                      gsem[b]).wait()
                pltpu.sync_copy(rows[b], acc.at[didx.at[j]], add=True)
                if j + NBUF < J:
                    pltpu.async_copy(ytab.at[sidx.at[j + NBUF]], rows[b],
                                     gsem[b])
            return 0

        lax.fori_loop(0, NG, group, 0)
        plsc.subcore_barrier()
        pltpu.sync_copy(
            acc.at[pl.ds(s * RT, RT)], o.at[c, pl.ds(s * RT, RT)]
        )
        if h == 0:
            plsc.subcore_barrier()


_layer = pl.kernel(
    _layer_body,
    out_type=[
        jax.ShapeDtypeStruct((NC, R, H), jnp.float32),
        jax.ShapeDtypeStruct((NC, R, H), jnp.float32),
    ],
    mesh=_mesh,
    scratch_types=[
        pltpu.VMEM((J, CL), jnp.int32),
        pltpu.VMEM((J, CL), jnp.int32),
        pltpu.VMEM((CL, H), jnp.float32),
        pltpu.VMEM_SHARED((R, H), jnp.float32),
    ]
    + [pltpu.VMEM((CL, H), jnp.float32)] * NBUF
    + [pltpu.SemaphoreType.DMA] * NBUF,
    compiler_params=_params,
)


@jax.jit
def kernel(edge_index, user_emb, item_emb):
    ei = edge_index.astype(jnp.int32)
    eidx = jnp.pad(ei, ((0, 0), (0, EP - E)), constant_values=TRASH)
    eidx = eidx.reshape(NC, EP // CL, CL)

    cnt = _hist(eidx)
    degu, degi = cnt[0, :NU], cnt[1, :NU]
    disu = jnp.where(degu > 0, lax.rsqrt(degu), 0.0)[:, None]
    disi = jnp.where(degi > 0, lax.rsqrt(degi), 0.0)[:, None]
    d2pad = (
        jnp.zeros((NC, R, 1), jnp.float32)
        .at[0, :NU].set(disu * disu)
        .at[1, :NU].set(disi * disi)
    )

    xu = user_emb * disu
    xi = item_emb * disi
    ztab = jnp.zeros((NC, R, H), jnp.float32)
    w0 = ztab.at[0, :NU].set(xu[:, :H]).at[1, :NU].set(xi[:, :H])
    w1 = ztab.at[0, :NU].set(xu[:, H:]).at[1, :NU].set(xi[:, H:])

    ws0 = jnp.zeros((NC, R, H), jnp.float32)
    ws1 = jnp.zeros((NC, R, H), jnp.float32)
    for l in range(3):
        o0, o1 = _layer(eidx, w0, w1)
        w0 = o0 * d2pad
        w1 = o1 * d2pad
        ws0 = ws0 + w0
        ws1 = ws1 + w1

    squ = jnp.sqrt(degu)[:, None]
    sqi = jnp.sqrt(degi)[:, None]
    user_final = (
        user_emb + squ * jnp.concatenate([ws0[0, :NU], ws1[0, :NU]], axis=1)
    ) * 0.25
    item_final = (
        item_emb + sqi * jnp.concatenate([ws0[1, :NU], ws1[1, :NU]], axis=1)
    ) * 0.25
    return user_final, item_final


# speedup vs baseline: 1.1211x; 1.1211x over previous
import jax
import jax.numpy as jnp
from jax import lax
from jax.experimental import pallas as pl
from jax.experimental.pallas import tpu as pltpu
from jax.experimental.pallas import tpu_sc as plsc

NU = 50000
H = 32
E = 800000
NC, NS, LN = 2, 16, 16
CL = 128
EP = 802816
G = EP // NS // CL
J = 8
NG = G // J
R = 51200
RT = R // NS
TRASH = R - 1
NBUF = 4

_mesh = plsc.VectorSubcoreMesh(
    core_axis_name="c", subcore_axis_name="s", num_cores=NC, num_subcores=NS
)
_params = pltpu.CompilerParams(use_tc_tiling_on_sc=False)


def _zero_f32(ref, nrows, ncols):
    z = jnp.zeros((LN,), jnp.float32)

    def body(r, _):
        for c0 in range(0, ncols, LN):
            ref[r, pl.ds(c0, LN)] = z
        return 0

    lax.fori_loop(0, nrows, body, 0)


def _zero_f32_1d(ref, n):
    z = jnp.zeros((LN,), jnp.float32)

    def body(r, _):
        ref[pl.ds(r * LN, LN)] = z
        return 0

    lax.fori_loop(0, n // LN, body, 0)


def _hist_body(eidx, cnt, didx, ones, zbuf, acc):
    c = lax.axis_index("c")
    s = lax.axis_index("s")
    one = jnp.ones((LN,), jnp.float32)
    for c0 in range(0, CL, LN):
        ones[pl.ds(c0, LN)] = one
    _zero_f32_1d(zbuf, RT)
    pltpu.sync_copy(zbuf, acc.at[pl.ds(s * RT, RT)])
    plsc.subcore_barrier()

    def group(g, _):
        base = s * G + g * J
        pltpu.sync_copy(eidx.at[c, pl.ds(base, J)], didx)
        for j in range(J):
            pltpu.sync_copy(ones, acc.at[didx.at[j]], add=True)
        return 0

    lax.fori_loop(0, NG, group, 0)
    plsc.subcore_barrier()
    pltpu.sync_copy(acc.at[pl.ds(s * RT, RT)], cnt.at[c, pl.ds(s * RT, RT)])


_hist = pl.kernel(
    _hist_body,
    out_type=jax.ShapeDtypeStruct((NC, R), jnp.float32),
    mesh=_mesh,
    scratch_types=[
        pltpu.VMEM((J, CL), jnp.int32),
        pltpu.VMEM((CL,), jnp.float32),
        pltpu.VMEM((RT,), jnp.float32),
        pltpu.VMEM_SHARED((R,), jnp.float32),
    ],
    compiler_params=_params,
)


def _layer_body(eidx, w0, w1, o0, o1, sidx, didx, zbuf, acc, isem_s, isem_d,
                *rows_sems):
    rows = rows_sems[:NBUF]
    gsem = rows_sems[NBUF:]
    c = lax.axis_index("c")
    s = lax.axis_index("s")
    notc = 1 - c
    _zero_f32(zbuf, CL, H)
    for h, (w, o) in enumerate(((w0, o0), (w1, o1))):
        ytab = w.at[notc]
        for k in range(RT // CL):
            pltpu.sync_copy(zbuf, acc.at[pl.ds(s * RT + k * CL, CL)])
        plsc.subcore_barrier()

        pltpu.async_copy(eidx.at[notc, pl.ds(s * G, J)],
                         sidx.at[pl.ds(0, J)], isem_s)
        pltpu.async_copy(eidx.at[c, pl.ds(s * G, J)],
                         didx.at[pl.ds(0, J)], isem_d)

        def group(g, _):
            base = s * G + g * J
            par = (g % 2) * J
            nxt = J - par
            basen = jnp.minimum(base + J, NS * G - J)
            pltpu.make_async_copy(eidx.at[notc, pl.ds(base, J)],
                                  sidx.at[pl.ds(par, J)], isem_s).wait()
            pltpu.make_async_copy(eidx.at[c, pl.ds(base, J)],
                                  didx.at[pl.ds(par, J)], isem_d).wait()
            pltpu.async_copy(eidx.at[notc, pl.ds(basen, J)],
                             sidx.at[pl.ds(nxt, J)], isem_s)
            pltpu.async_copy(eidx.at[c, pl.ds(basen, J)],
                             didx.at[pl.ds(nxt, J)], isem_d)
            for b in range(NBUF):
                pltpu.async_copy(ytab.at[sidx.at[par + b]], rows[b],
                                 gsem[b])
            for j in range(J):
                b = j % NBUF
                pltpu.make_async_copy(ytab.at[sidx.at[par + j]], rows[b],
                                      gsem[b]).wait()
                pltpu.sync_copy(rows[b], acc.at[didx.at[par + j]], add=True)
                if j + NBUF < J:
                    pltpu.async_copy(ytab.at[sidx.at[par + j + NBUF]],
                                     rows[b], gsem[b])
            return 0

        lax.fori_loop(0, NG, group, 0)
        pltpu.make_async_copy(eidx.at[notc, pl.ds(s * G, J)],
                              sidx.at[pl.ds(0, J)], isem_s).wait()
        pltpu.make_async_copy(eidx.at[c, pl.ds(s * G, J)],
                              didx.at[pl.ds(0, J)], isem_d).wait()
        plsc.subcore_barrier()
        pltpu.sync_copy(
            acc.at[pl.ds(s * RT, RT)], o.at[c, pl.ds(s * RT, RT)]
        )
        if h == 0:
            plsc.subcore_barrier()


_layer = pl.kernel(
    _layer_body,
    out_type=[
        jax.ShapeDtypeStruct((NC, R, H), jnp.float32),
        jax.ShapeDtypeStruct((NC, R, H), jnp.float32),
    ],
    mesh=_mesh,
    scratch_types=[
        pltpu.VMEM((2 * J, CL), jnp.int32),
        pltpu.VMEM((2 * J, CL), jnp.int32),
        pltpu.VMEM((CL, H), jnp.float32),
        pltpu.VMEM_SHARED((R, H), jnp.float32),
        pltpu.SemaphoreType.DMA,
        pltpu.SemaphoreType.DMA,
    ]
    + [pltpu.VMEM((CL, H), jnp.float32)] * NBUF
    + [pltpu.SemaphoreType.DMA] * NBUF,
    compiler_params=_params,
)


@jax.jit
def kernel(edge_index, user_emb, item_emb):
    ei = edge_index.astype(jnp.int32)
    eidx = jnp.pad(ei, ((0, 0), (0, EP - E)), constant_values=TRASH)
    eidx = eidx.reshape(NC, EP // CL, CL)

    cnt = _hist(eidx)
    degu, degi = cnt[0, :NU], cnt[1, :NU]
    disu = jnp.where(degu > 0, lax.rsqrt(degu), 0.0)[:, None]
    disi = jnp.where(degi > 0, lax.rsqrt(degi), 0.0)[:, None]
    d2pad = (
        jnp.zeros((NC, R, 1), jnp.float32)
        .at[0, :NU].set(disu * disu)
        .at[1, :NU].set(disi * disi)
    )

    xu = user_emb * disu
    xi = item_emb * disi
    ztab = jnp.zeros((NC, R, H), jnp.float32)
    w0 = ztab.at[0, :NU].set(xu[:, :H]).at[1, :NU].set(xi[:, :H])
    w1 = ztab.at[0, :NU].set(xu[:, H:]).at[1, :NU].set(xi[:, H:])

    ws0 = jnp.zeros((NC, R, H), jnp.float32)
    ws1 = jnp.zeros((NC, R, H), jnp.float32)
    for l in range(3):
        o0, o1 = _layer(eidx, w0, w1)
        w0 = o0 * d2pad
        w1 = o1 * d2pad
        ws0 = ws0 + w0
        ws1 = ws1 + w1

    squ = jnp.sqrt(degu)[:, None]
    sqi = jnp.sqrt(degi)[:, None]
    user_final = (
        user_emb + squ * jnp.concatenate([ws0[0, :NU], ws1[0, :NU]], axis=1)
    ) * 0.25
    item_final = (
        item_emb + sqi * jnp.concatenate([ws0[1, :NU], ws1[1, :NU]], axis=1)
    ) * 0.25
    return user_final, item_final


# speedup vs baseline: 1.2454x; 1.1109x over previous
import jax
import jax.numpy as jnp
from jax import lax
from jax.experimental import pallas as pl
from jax.experimental.pallas import tpu as pltpu
from jax.experimental.pallas import tpu_sc as plsc

NU = 50000
H = 32
E = 800000
NC, NS, LN = 2, 16, 16
CL = 128
EP = 802816
G = EP // NS // CL
J = 8
NG = G // J
R = 51200
RT = R // NS
TRASH = R - 1
NBUF = 4

_mesh = plsc.VectorSubcoreMesh(
    core_axis_name="c", subcore_axis_name="s", num_cores=NC, num_subcores=NS
)
_params = pltpu.CompilerParams(use_tc_tiling_on_sc=False)


def _zero_f32(ref, nrows, ncols):
    z = jnp.zeros((LN,), jnp.float32)

    def body(r, _):
        for c0 in range(0, ncols, LN):
            ref[r, pl.ds(c0, LN)] = z
        return 0

    lax.fori_loop(0, nrows, body, 0)


def _zero_f32_1d(ref, n):
    z = jnp.zeros((LN,), jnp.float32)

    def body(r, _):
        ref[pl.ds(r * LN, LN)] = z
        return 0

    lax.fori_loop(0, n // LN, body, 0)


def _hist_body(eidx, cnt, didx, ones, zbuf, acc):
    c = lax.axis_index("c")
    s = lax.axis_index("s")
    one = jnp.ones((LN,), jnp.float32)
    for c0 in range(0, CL, LN):
        ones[pl.ds(c0, LN)] = one
    _zero_f32_1d(zbuf, RT)
    pltpu.sync_copy(zbuf, acc.at[pl.ds(s * RT, RT)])
    plsc.subcore_barrier()

    def group(g, _):
        base = s * G + g * J
        pltpu.sync_copy(eidx.at[c, pl.ds(base, J)], didx)
        for j in range(J):
            pltpu.sync_copy(ones, acc.at[didx.at[j]], add=True)
        return 0

    lax.fori_loop(0, NG, group, 0)
    plsc.subcore_barrier()
    pltpu.sync_copy(acc.at[pl.ds(s * RT, RT)], cnt.at[c, pl.ds(s * RT, RT)])


_hist = pl.kernel(
    _hist_body,
    out_type=jax.ShapeDtypeStruct((NC, R), jnp.float32),
    mesh=_mesh,
    scratch_types=[
        pltpu.VMEM((J, CL), jnp.int32),
        pltpu.VMEM((CL,), jnp.float32),
        pltpu.VMEM((RT,), jnp.float32),
        pltpu.VMEM_SHARED((R,), jnp.float32),
    ],
    compiler_params=_params,
)


def _layer_body(eidx, w0, w1, o0, o1, sidx, didx, zbuf, acc, isem_s, isem_d,
                *rows_sems):
    rows = rows_sems[:NBUF]
    gsem = rows_sems[NBUF:]
    c = lax.axis_index("c")
    s = lax.axis_index("s")
    notc = 1 - c
    _zero_f32(zbuf, CL, H)
    for h, (w, o) in enumerate(((w0, o0), (w1, o1))):
        ytab = w.at[notc]
        for k in range(RT // CL):
            pltpu.sync_copy(zbuf, acc.at[pl.ds(s * RT + k * CL, CL)])
        plsc.subcore_barrier()

        pltpu.async_copy(eidx.at[notc, pl.ds(s * G, J)],
                         sidx.at[pl.ds(0, J)], isem_s)
        pltpu.async_copy(eidx.at[c, pl.ds(s * G, J)],
                         didx.at[pl.ds(0, J)], isem_d)
        pltpu.make_async_copy(eidx.at[notc, pl.ds(s * G, J)],
                              sidx.at[pl.ds(0, J)], isem_s).wait()
        pltpu.make_async_copy(eidx.at[c, pl.ds(s * G, J)],
                              didx.at[pl.ds(0, J)], isem_d).wait()
        pltpu.async_copy(eidx.at[notc, pl.ds(s * G + J, J)],
                         sidx.at[pl.ds(J, J)], isem_s)
        pltpu.async_copy(eidx.at[c, pl.ds(s * G + J, J)],
                         didx.at[pl.ds(J, J)], isem_d)
        for b in range(NBUF):
            pltpu.async_copy(ytab.at[sidx.at[b]], rows[b], gsem[b])

        def group(g, _):
            base = s * G + g * J
            par = (g % 2) * J
            nxt = J - par
            for j in range(J):
                b = j % NBUF
                pltpu.make_async_copy(ytab.at[sidx.at[par + j]], rows[b],
                                      gsem[b]).wait()
                pltpu.sync_copy(rows[b], acc.at[didx.at[par + j]], add=True)
                if j + NBUF < J:
                    pltpu.async_copy(ytab.at[sidx.at[par + j + NBUF]],
                                     rows[b], gsem[b])
                else:
                    if j == J - NBUF:
                        pltpu.make_async_copy(
                            eidx.at[notc, pl.ds(base, J)],
                            sidx.at[pl.ds(nxt, J)], isem_s).wait()
                        pltpu.make_async_copy(
                            eidx.at[c, pl.ds(base, J)],
                            didx.at[pl.ds(nxt, J)], isem_d).wait()
                    pltpu.async_copy(
                        ytab.at[sidx.at[nxt + j + NBUF - J]], rows[b],
                        gsem[b])
            basen = jnp.minimum(base + 2 * J, NS * G - J)
            pltpu.async_copy(eidx.at[notc, pl.ds(basen, J)],
                             sidx.at[pl.ds(par, J)], isem_s)
            pltpu.async_copy(eidx.at[c, pl.ds(basen, J)],
                             didx.at[pl.ds(par, J)], isem_d)
            return 0

        lax.fori_loop(0, NG, group, 0)
        for b in range(NBUF):
            pltpu.make_async_copy(ytab.at[sidx.at[b]], rows[b],
                                  gsem[b]).wait()
        pltpu.make_async_copy(eidx.at[notc, pl.ds(s * G, J)],
                              sidx.at[pl.ds(0, J)], isem_s).wait()
        pltpu.make_async_copy(eidx.at[c, pl.ds(s * G, J)],
                              didx.at[pl.ds(0, J)], isem_d).wait()
        plsc.subcore_barrier()
        pltpu.sync_copy(
            acc.at[pl.ds(s * RT, RT)], o.at[c, pl.ds(s * RT, RT)]
        )
        if h == 0:
            plsc.subcore_barrier()


_layer = pl.kernel(
    _layer_body,
    out_type=[
        jax.ShapeDtypeStruct((NC, R, H), jnp.float32),
        jax.ShapeDtypeStruct((NC, R, H), jnp.float32),
    ],
    mesh=_mesh,
    scratch_types=[
        pltpu.VMEM((2 * J, CL), jnp.int32),
        pltpu.VMEM((2 * J, CL), jnp.int32),
        pltpu.VMEM((CL, H), jnp.float32),
        pltpu.VMEM_SHARED((R, H), jnp.float32),
        pltpu.SemaphoreType.DMA,
        pltpu.SemaphoreType.DMA,
    ]
    + [pltpu.VMEM((CL, H), jnp.float32)] * NBUF
    + [pltpu.SemaphoreType.DMA] * NBUF,
    compiler_params=_params,
)


@jax.jit
def kernel(edge_index, user_emb, item_emb):
    ei = edge_index.astype(jnp.int32)
    eidx = jnp.pad(ei, ((0, 0), (0, EP - E)), constant_values=TRASH)
    eidx = eidx.reshape(NC, EP // CL, CL)

    cnt = _hist(eidx)
    degu, degi = cnt[0, :NU], cnt[1, :NU]
    disu = jnp.where(degu > 0, lax.rsqrt(degu), 0.0)[:, None]
    disi = jnp.where(degi > 0, lax.rsqrt(degi), 0.0)[:, None]
    d2pad = (
        jnp.zeros((NC, R, 1), jnp.float32)
        .at[0, :NU].set(disu * disu)
        .at[1, :NU].set(disi * disi)
    )

    xu = user_emb * disu
    xi = item_emb * disi
    ztab = jnp.zeros((NC, R, H), jnp.float32)
    w0 = ztab.at[0, :NU].set(xu[:, :H]).at[1, :NU].set(xi[:, :H])
    w1 = ztab.at[0, :NU].set(xu[:, H:]).at[1, :NU].set(xi[:, H:])

    ws0 = jnp.zeros((NC, R, H), jnp.float32)
    ws1 = jnp.zeros((NC, R, H), jnp.float32)
    for l in range(3):
        o0, o1 = _layer(eidx, w0, w1)
        w0 = o0 * d2pad
        w1 = o1 * d2pad
        ws0 = ws0 + w0
        ws1 = ws1 + w1

    squ = jnp.sqrt(degu)[:, None]
    sqi = jnp.sqrt(degi)[:, None]
    user_final = (
        user_emb + squ * jnp.concatenate([ws0[0, :NU], ws1[0, :NU]], axis=1)
    ) * 0.25
    item_final = (
        item_emb + sqi * jnp.concatenate([ws0[1, :NU], ws1[1, :NU]], axis=1)
    ) * 0.25
    return user_final, item_final
